# baseline (speedup 1.0000x reference)
import functools
import math

import jax
import jax.numpy as jnp
from jax import lax
from jax.experimental import pallas as pl
from jax.experimental.pallas import tpu as pltpu
from jax.experimental.pallas import tpu_sc as plsc

D_MODEL = 64
SCALE = math.sqrt(D_MODEL)

_INFO = plsc.get_sparse_core_info()
NC = _INFO.num_cores
NS = _INFO.num_subcores
NW = NC * NS
CHUNK = 128


@functools.lru_cache(maxsize=None)
def _build(n_blocks):
    mesh = plsc.VectorSubcoreMesh(core_axis_name="c", subcore_axis_name="s")

    @functools.partial(
        pl.kernel,
        mesh=mesh,
        out_type=jax.ShapeDtypeStruct((NW, n_blocks, CHUNK, D_MODEL), jnp.float32),
        scratch_types=[
            pltpu.VMEM((n_blocks, CHUNK), jnp.int32),
            pltpu.VMEM((CHUNK, D_MODEL), jnp.float32),
            pltpu.VMEM((CHUNK, D_MODEL), jnp.float32),
            pltpu.SemaphoreType.DMA,
        ],
        compiler_params=pltpu.CompilerParams(use_tc_tiling_on_sc=False),
    )
    def emb(x_hbm, table_hbm, out_hbm, idx_v, rows_v, scaled_v, gsem):
        wid = lax.axis_index("s") * NC + lax.axis_index("c")
        pltpu.sync_copy(x_hbm.at[wid], idx_v)

        def step(j, _):
            pltpu.async_copy(table_hbm.at[idx_v.at[j]], rows_v, gsem).wait()

            def mul_row(r, _):
                for q in range(D_MODEL // 16):
                    seg = pl.ds(q * 16, 16)
                    scaled_v[r, seg] = rows_v[r, seg] * SCALE
                return ()

            lax.fori_loop(0, CHUNK, mul_row, ())
            pltpu.sync_copy(scaled_v, out_hbm.at[wid, j])
            return ()

        lax.fori_loop(0, n_blocks, step, ())

    return emb


def kernel(x, table):
    b, s = x.shape
    total = b * s
    per_w = total // NW
    n_blocks = per_w // CHUNK
    xw = x.reshape(NW, n_blocks, CHUNK).astype(jnp.int32)
    out = _build(n_blocks)(xw, table)
    return out.reshape(b, s, D_MODEL)


# speedup vs baseline: 1.1898x; 1.1898x over previous
import functools
import math

import jax
import jax.numpy as jnp
from jax import lax
from jax.experimental import pallas as pl
from jax.experimental.pallas import tpu as pltpu
from jax.experimental.pallas import tpu_sc as plsc

D_MODEL = 64
SCALE = math.sqrt(D_MODEL)

_INFO = plsc.get_sparse_core_info()
NC = _INFO.num_cores
NS = _INFO.num_subcores
NW = NC * NS
CHUNK = 128
NBUF = 4


@functools.lru_cache(maxsize=None)
def _build(n_blocks):
    mesh = plsc.VectorSubcoreMesh(core_axis_name="c", subcore_axis_name="s")

    @functools.partial(
        pl.kernel,
        mesh=mesh,
        out_type=jax.ShapeDtypeStruct((NW, n_blocks, CHUNK, D_MODEL), jnp.float32),
        scratch_types=[
            pltpu.VMEM((n_blocks, CHUNK), jnp.int32),
            [pltpu.VMEM((CHUNK, D_MODEL), jnp.float32) for _ in range(NBUF)],
            [pltpu.VMEM((CHUNK, D_MODEL), jnp.float32) for _ in range(NBUF)],
            [pltpu.SemaphoreType.DMA for _ in range(NBUF)],
            [pltpu.SemaphoreType.DMA for _ in range(NBUF)],
        ],
        compiler_params=pltpu.CompilerParams(use_tc_tiling_on_sc=False),
    )
    def emb(x_hbm, table_hbm, out_hbm, idx_v, ins, outs, gsems, ssems):
        wid = lax.axis_index("s") * NC + lax.axis_index("c")
        pltpu.sync_copy(x_hbm.at[wid], idx_v)

        for b in range(NBUF):
            pltpu.async_copy(table_hbm.at[idx_v.at[b]], ins[b], gsems[b])

        def step(j, _):
            for b in range(NBUF):
                blk = j + b
                pltpu.make_async_copy(
                    table_hbm.at[idx_v.at[blk]], ins[b], gsems[b]
                ).wait()
                @pl.when(j > 0)
                def _():
                    pltpu.make_async_copy(
                        outs[b], out_hbm.at[wid, blk], ssems[b]
                    ).wait()

                def mul4(r4, _):
                    r = r4 * 4
                    for rr in range(4):
                        for q in range(D_MODEL // 16):
                            seg = pl.ds(q * 16, 16)
                            outs[b][r + rr, seg] = ins[b][r + rr, seg] * SCALE
                    return ()

                lax.fori_loop(0, CHUNK // 4, mul4, ())
                pltpu.async_copy(outs[b], out_hbm.at[wid, blk], ssems[b])

                @pl.when(j + NBUF < n_blocks)
                def _():
                    pltpu.async_copy(
                        table_hbm.at[idx_v.at[blk + NBUF]], ins[b], gsems[b]
                    )
            return ()

        lax.fori_loop(0, n_blocks // NBUF, lambda i, c: step(i * NBUF, c), ())

        for b in range(NBUF):
            pltpu.make_async_copy(
                outs[b], out_hbm.at[wid, n_blocks - NBUF + b], ssems[b]
            ).wait()

    return emb


def kernel(x, table):
    b, s = x.shape
    total = b * s
    per_w = total // NW
    n_blocks = per_w // CHUNK
    xw = x.reshape(NW, n_blocks, CHUNK).astype(jnp.int32)
    out = _build(n_blocks)(xw, table)
    return out.reshape(b, s, D_MODEL)


# speedup vs baseline: 1.8134x; 1.5241x over previous
import functools

import jax
import jax.numpy as jnp
from jax import lax
from jax.experimental import pallas as pl
from jax.experimental.pallas import tpu as pltpu
from jax.experimental.pallas import tpu_sc as plsc

D_MODEL = 64
SCALE = 8.0

_INFO = plsc.get_sparse_core_info()
NC = _INFO.num_cores
NS = _INFO.num_subcores
NW = NC * NS
CHUNK = 64
NBUF = 4
KCOL = 8192


@functools.lru_cache(maxsize=None)
def _build_dup(vocab):
    grid = (vocab + KCOL - 1) // KCOL
    vpad = grid * KCOL

    def dup(tt_ref, t4_ref):
        y = tt_ref[...].T * SCALE
        t4_ref[...] = jnp.concatenate([y, y], axis=1)

    return pl.pallas_call(
        dup,
        grid=(grid,),
        in_specs=[pl.BlockSpec((D_MODEL, KCOL), lambda g: (0, g))],
        out_specs=pl.BlockSpec((KCOL, 2 * D_MODEL), lambda g: (g, 0)),
        out_shape=jax.ShapeDtypeStruct((vpad, 2 * D_MODEL), jnp.float32),
    )


@functools.lru_cache(maxsize=None)
def _build_gather(n_blocks, vpad):
    mesh = plsc.VectorSubcoreMesh(core_axis_name="c", subcore_axis_name="s")

    @functools.partial(
        pl.kernel,
        mesh=mesh,
        out_type=jax.ShapeDtypeStruct(
            (NW, n_blocks, CHUNK, D_MODEL), jnp.float32
        ),
        scratch_types=[
            pltpu.VMEM((n_blocks, CHUNK), jnp.int32),
            [pltpu.VMEM((CHUNK, 2 * D_MODEL), jnp.float32) for _ in range(NBUF)],
            [pltpu.VMEM((CHUNK, D_MODEL), jnp.float32) for _ in range(NBUF)],
            [pltpu.SemaphoreType.DMA for _ in range(NBUF)],
            [pltpu.SemaphoreType.DMA for _ in range(NBUF)],
        ],
    )
    def gather(x_hbm, t4_hbm, out_hbm, idx_v, ins, outs, gsems, ssems):
        wid = lax.axis_index("s") * NC + lax.axis_index("c")
        pltpu.sync_copy(x_hbm.at[wid], idx_v)

        for b in range(NBUF):
            pltpu.async_copy(t4_hbm.at[idx_v.at[b]], ins[b], gsems[b])

        def step(j, _):
            for b in range(NBUF):
                blk = j + b
                pltpu.make_async_copy(
                    t4_hbm.at[idx_v.at[blk]], ins[b], gsems[b]
                ).wait()

                @pl.when(j > 0)
                def _():
                    pltpu.make_async_copy(
                        outs[b], out_hbm.at[wid, blk], ssems[b]
                    ).wait()

                def crow(r4, _):
                    r = r4 * 4
                    for rr in range(4):
                        for q in range(D_MODEL // 16):
                            seg = pl.ds(q * 16, 16)
                            outs[b][r + rr, seg] = ins[b][r + rr, seg]
                    return ()

                lax.fori_loop(0, CHUNK // 4, crow, ())

                pltpu.async_copy(outs[b], out_hbm.at[wid, blk], ssems[b])

                @pl.when(j + NBUF < n_blocks)
                def _():
                    pltpu.async_copy(
                        t4_hbm.at[idx_v.at[blk + NBUF]], ins[b], gsems[b]
                    )
            return ()

        lax.fori_loop(0, n_blocks // NBUF, lambda i, c: step(i * NBUF, c), ())

        for b in range(NBUF):
            pltpu.make_async_copy(
                outs[b], out_hbm.at[wid, n_blocks - NBUF + b], ssems[b]
            ).wait()

    return gather


def kernel(x, table):
    n_batch, seq_len = x.shape
    vocab, d = table.shape
    total = n_batch * seq_len
    per_w = total // NW
    n_blocks = per_w // CHUNK
    t4 = _build_dup(vocab)(table.T)
    xw = x.reshape(NW, n_blocks, CHUNK).astype(jnp.int32)
    out = _build_gather(n_blocks, t4.shape[0])(xw, t4)
    return out.reshape(n_batch, seq_len, D_MODEL)


# speedup vs baseline: 1.8150x; 1.0009x over previous
import functools

import jax
import jax.numpy as jnp
from jax import lax
from jax.experimental import pallas as pl
from jax.experimental.pallas import tpu as pltpu
from jax.experimental.pallas import tpu_sc as plsc

D_MODEL = 64
SCALE = 8.0

_INFO = plsc.get_sparse_core_info()
NC = _INFO.num_cores
NS = _INFO.num_subcores
NW = NC * NS
CHUNK = 128
NBUF = 2
KCOL = 8192


@functools.lru_cache(maxsize=None)
def _build_dup(vocab):
    grid = (vocab + KCOL - 1) // KCOL
    vpad = grid * KCOL

    def dup(tt_ref, t4_ref):
        y = tt_ref[...].T * SCALE
        t4_ref[...] = jnp.concatenate([y, y], axis=1)

    return pl.pallas_call(
        dup,
        grid=(grid,),
        in_specs=[pl.BlockSpec((D_MODEL, KCOL), lambda g: (0, g))],
        out_specs=pl.BlockSpec((KCOL, 2 * D_MODEL), lambda g: (g, 0)),
        out_shape=jax.ShapeDtypeStruct((vpad, 2 * D_MODEL), jnp.float32),
    )


@functools.lru_cache(maxsize=None)
def _build_gather(n_blocks, vpad):
    mesh = plsc.VectorSubcoreMesh(core_axis_name="c", subcore_axis_name="s")

    @functools.partial(
        pl.kernel,
        mesh=mesh,
        out_type=jax.ShapeDtypeStruct(
            (NW, n_blocks, CHUNK, D_MODEL), jnp.float32
        ),
        scratch_types=[
            pltpu.VMEM((n_blocks, CHUNK), jnp.int32),
            [pltpu.VMEM((CHUNK, 2 * D_MODEL), jnp.float32) for _ in range(NBUF)],
            [pltpu.VMEM((CHUNK, D_MODEL), jnp.float32) for _ in range(NBUF)],
            [pltpu.SemaphoreType.DMA for _ in range(NBUF)],
            [pltpu.SemaphoreType.DMA for _ in range(NBUF)],
        ],
    )
    def gather(x_hbm, t4_hbm, out_hbm, idx_v, ins, outs, gsems, ssems):
        wid = lax.axis_index("s") * NC + lax.axis_index("c")
        pltpu.sync_copy(x_hbm.at[wid], idx_v)

        for b in range(NBUF):
            pltpu.async_copy(t4_hbm.at[idx_v.at[b]], ins[b], gsems[b])

        def step(j, _):
            for b in range(NBUF):
                blk = j + b
                pltpu.make_async_copy(
                    t4_hbm.at[idx_v.at[blk]], ins[b], gsems[b]
                ).wait()

                @pl.when(j > 0)
                def _():
                    pltpu.make_async_copy(
                        outs[b], out_hbm.at[wid, blk], ssems[b]
                    ).wait()

                def crow(r4, _):
                    r = r4 * 4
                    for rr in range(4):
                        for q in range(D_MODEL // 16):
                            seg = pl.ds(q * 16, 16)
                            outs[b][r + rr, seg] = ins[b][r + rr, seg]
                    return ()

                lax.fori_loop(0, CHUNK // 4, crow, ())

                pltpu.async_copy(outs[b], out_hbm.at[wid, blk], ssems[b])

                @pl.when(j + NBUF < n_blocks)
                def _():
                    pltpu.async_copy(
                        t4_hbm.at[idx_v.at[blk + NBUF]], ins[b], gsems[b]
                    )
            return ()

        lax.fori_loop(0, n_blocks // NBUF, lambda i, c: step(i * NBUF, c), ())

        for b in range(NBUF):
            pltpu.make_async_copy(
                outs[b], out_hbm.at[wid, n_blocks - NBUF + b], ssems[b]
            ).wait()

    return gather


def kernel(x, table):
    n_batch, seq_len = x.shape
    vocab, d = table.shape
    total = n_batch * seq_len
    per_w = total // NW
    n_blocks = per_w // CHUNK
    t4 = _build_dup(vocab)(table.T)
    xw = x.reshape(NW, n_blocks, CHUNK).astype(jnp.int32)
    out = _build_gather(n_blocks, t4.shape[0])(xw, t4)
    return out.reshape(n_batch, seq_len, D_MODEL)


# speedup vs baseline: 1.8851x; 1.0386x over previous
import functools

import jax
import jax.numpy as jnp
from jax import lax
from jax.experimental import pallas as pl
from jax.experimental.pallas import tpu as pltpu
from jax.experimental.pallas import tpu_sc as plsc

D_MODEL = 64
SCALE = 8.0

_INFO = plsc.get_sparse_core_info()
NC = _INFO.num_cores
NS = _INFO.num_subcores
NW = NC * NS
CHUNK = 128
NBUF = 2
KCOL = 16384


@functools.lru_cache(maxsize=None)
def _build_dup(vocab):
    grid = (vocab + KCOL - 1) // KCOL
    vpad = grid * KCOL

    def dup(tt_ref, t4_ref):
        y = tt_ref[...].T
        t4_ref[...] = jnp.concatenate([y, y], axis=1)

    return pl.pallas_call(
        dup,
        grid=(grid,),
        in_specs=[pl.BlockSpec((D_MODEL, KCOL), lambda g: (0, g))],
        out_specs=pl.BlockSpec((KCOL, 2 * D_MODEL), lambda g: (g, 0)),
        out_shape=jax.ShapeDtypeStruct((vpad, 2 * D_MODEL), jnp.float32),
    )


@functools.lru_cache(maxsize=None)
def _build_gather(n_blocks, vpad):
    mesh = plsc.VectorSubcoreMesh(core_axis_name="c", subcore_axis_name="s")

    @functools.partial(
        pl.kernel,
        mesh=mesh,
        out_type=jax.ShapeDtypeStruct(
            (NW, n_blocks, CHUNK, D_MODEL), jnp.float32
        ),
        scratch_types=[
            pltpu.VMEM((n_blocks, CHUNK), jnp.int32),
            [pltpu.VMEM((CHUNK, 2 * D_MODEL), jnp.float32) for _ in range(NBUF)],
            [pltpu.VMEM((CHUNK, D_MODEL), jnp.float32) for _ in range(NBUF)],
            [pltpu.SemaphoreType.DMA for _ in range(NBUF)],
            [pltpu.SemaphoreType.DMA for _ in range(NBUF)],
        ],
    )
    def gather(x_hbm, t4_hbm, out_hbm, idx_v, ins, outs, gsems, ssems):
        wid = lax.axis_index("s") * NC + lax.axis_index("c")
        pltpu.sync_copy(x_hbm.at[wid], idx_v)

        for b in range(NBUF):
            pltpu.async_copy(t4_hbm.at[idx_v.at[b]], ins[b], gsems[b])

        def step(j, _):
            for b in range(NBUF):
                blk = j + b
                pltpu.make_async_copy(
                    t4_hbm.at[idx_v.at[blk]], ins[b], gsems[b]
                ).wait()

                @pl.when(j > 0)
                def _():
                    pltpu.make_async_copy(
                        outs[b], out_hbm.at[wid, blk], ssems[b]
                    ).wait()

                def crow(r4, _):
                    r = r4 * 4
                    for rr in range(4):
                        for q in range(D_MODEL // 16):
                            seg = pl.ds(q * 16, 16)
                            outs[b][r + rr, seg] = ins[b][r + rr, seg] * SCALE
                    return ()

                lax.fori_loop(0, CHUNK // 4, crow, ())

                pltpu.async_copy(outs[b], out_hbm.at[wid, blk], ssems[b])

                @pl.when(j + NBUF < n_blocks)
                def _():
                    pltpu.async_copy(
                        t4_hbm.at[idx_v.at[blk + NBUF]], ins[b], gsems[b]
                    )
            return ()

        lax.fori_loop(0, n_blocks // NBUF, lambda i, c: step(i * NBUF, c), ())

        for b in range(NBUF):
            pltpu.make_async_copy(
                outs[b], out_hbm.at[wid, n_blocks - NBUF + b], ssems[b]
            ).wait()

    return gather


def kernel(x, table):
    n_batch, seq_len = x.shape
    vocab, d = table.shape
    total = n_batch * seq_len
    per_w = total // NW
    n_blocks = per_w // CHUNK
    t4 = _build_dup(vocab)(table.T)
    xw = x.reshape(NW, n_blocks, CHUNK).astype(jnp.int32)
    out = _build_gather(n_blocks, t4.shape[0])(xw, t4)
    return out.reshape(n_batch, seq_len, D_MODEL)


# speedup vs baseline: 1.9118x; 1.0141x over previous
import functools

import jax
import jax.numpy as jnp
from jax import lax
from jax.experimental import pallas as pl
from jax.experimental.pallas import tpu as pltpu
from jax.experimental.pallas import tpu_sc as plsc

D_MODEL = 64
SCALE = 8.0

_INFO = plsc.get_sparse_core_info()
NC = _INFO.num_cores
NS = _INFO.num_subcores
NW = NC * NS
CHUNK = 128
NBUF = 2
KCOL = 24576


@functools.lru_cache(maxsize=None)
def _build_dup(vocab):
    grid = (vocab + KCOL - 1) // KCOL
    vpad = grid * KCOL

    def dup(tt_ref, t4_ref):
        y = tt_ref[...].T
        t4_ref[...] = jnp.concatenate([y, y], axis=1)

    return pl.pallas_call(
        dup,
        grid=(grid,),
        in_specs=[pl.BlockSpec((D_MODEL, KCOL), lambda g: (0, g))],
        out_specs=pl.BlockSpec((KCOL, 2 * D_MODEL), lambda g: (g, 0)),
        out_shape=jax.ShapeDtypeStruct((vpad, 2 * D_MODEL), jnp.float32),
    )


@functools.lru_cache(maxsize=None)
def _build_gather(n_blocks, vpad):
    mesh = plsc.VectorSubcoreMesh(core_axis_name="c", subcore_axis_name="s")

    @functools.partial(
        pl.kernel,
        mesh=mesh,
        out_type=jax.ShapeDtypeStruct(
            (NW, n_blocks, CHUNK, D_MODEL), jnp.float32
        ),
        scratch_types=[
            pltpu.VMEM((n_blocks, CHUNK), jnp.int32),
            [pltpu.VMEM((CHUNK, 2 * D_MODEL), jnp.float32) for _ in range(NBUF)],
            [pltpu.VMEM((CHUNK, D_MODEL), jnp.float32) for _ in range(NBUF)],
            [pltpu.SemaphoreType.DMA for _ in range(NBUF)],
            [pltpu.SemaphoreType.DMA for _ in range(NBUF)],
        ],
    )
    def gather(x_hbm, t4_hbm, out_hbm, idx_v, ins, outs, gsems, ssems):
        wid = lax.axis_index("s") * NC + lax.axis_index("c")
        pltpu.sync_copy(x_hbm.at[wid], idx_v)

        for b in range(NBUF):
            pltpu.async_copy(t4_hbm.at[idx_v.at[b]], ins[b], gsems[b])

        def step(j, _):
            for b in range(NBUF):
                blk = j + b
                pltpu.make_async_copy(
                    t4_hbm.at[idx_v.at[blk]], ins[b], gsems[b]
                ).wait()

                @pl.when(j > 0)
                def _():
                    pltpu.make_async_copy(
                        outs[b], out_hbm.at[wid, blk], ssems[b]
                    ).wait()

                def crow(r4, _):
                    r = r4 * 4
                    for rr in range(4):
                        for q in range(D_MODEL // 16):
                            seg = pl.ds(q * 16, 16)
                            outs[b][r + rr, seg] = ins[b][r + rr, seg] * SCALE
                    return ()

                lax.fori_loop(0, CHUNK // 4, crow, ())

                pltpu.async_copy(outs[b], out_hbm.at[wid, blk], ssems[b])

                @pl.when(j + NBUF < n_blocks)
                def _():
                    pltpu.async_copy(
                        t4_hbm.at[idx_v.at[blk + NBUF]], ins[b], gsems[b]
                    )
            return ()

        lax.fori_loop(0, n_blocks // NBUF, lambda i, c: step(i * NBUF, c), ())

        for b in range(NBUF):
            pltpu.make_async_copy(
                outs[b], out_hbm.at[wid, n_blocks - NBUF + b], ssems[b]
            ).wait()

    return gather


def kernel(x, table):
    n_batch, seq_len = x.shape
    vocab, d = table.shape
    total = n_batch * seq_len
    per_w = total // NW
    n_blocks = per_w // CHUNK
    t4 = _build_dup(vocab)(table.T)
    xw = x.reshape(NW, n_blocks, CHUNK).astype(jnp.int32)
    out = _build_gather(n_blocks, t4.shape[0])(xw, t4)
    return out.reshape(n_batch, seq_len, D_MODEL)
